# baseline (speedup 1.0000x reference)
import functools

import jax
import jax.numpy as jnp
from jax import lax
from jax.experimental import pallas as pl
from jax.experimental.pallas import tpu as pltpu
from jax.experimental.pallas import tpu_sc as plsc

_NC = 2
_NS = 16
_ROWS = 400


_CH = 128
_CPW = 80
_RCHUNK = 80


_SINK = 80


def _segment_sum_sc(h, eidx):
    N, D = h.shape
    E = eidx.shape[1]
    NW = _NC * _NS
    n_ec = E // _CH
    assert n_ec == NW * _CPW and n_ec * _CH == E
    NP = N + _SINK
    n_oc = N // _RCHUNK
    assert n_oc * _RCHUNK == N and D % 16 == 0
    oc_full, oc_rem = divmod(n_oc, _NS)
    n_pairs = _CPW // 2
    mesh = plsc.VectorSubcoreMesh(core_axis_name="c", subcore_axis_name="s")

    @functools.partial(
        pl.kernel,
        out_type=jax.ShapeDtypeStruct((_NC, N, D), jnp.float32),
        mesh=mesh,
        scratch_types=[
            pltpu.VMEM((2, _CH), jnp.int32),
            pltpu.VMEM((2, _CH), jnp.int32),
            pltpu.VMEM((_CH, D), jnp.float32),
            pltpu.VMEM((_CH, D), jnp.float32),
            pltpu.VMEM((_RCHUNK, D), jnp.float32),
            pltpu.VMEM_SHARED((NP, D), jnp.float32),
            pltpu.SemaphoreType.DMA,
            pltpu.SemaphoreType.DMA,
        ],
    )
    def k(h_hbm, eidx_hbm, out_hbm, ib0, ib1, rows0, rows1, zbuf, agg_sh,
          sg0, sg1):
        cid = lax.axis_index("c")
        sid = lax.axis_index("s")
        wid = cid * _NS + sid

        def load_idx(c, ib):
            pltpu.sync_copy(eidx_hbm.at[:, pl.ds(c * _CH, _CH)], ib)

        def gather(ib, rows, sg):
            return pltpu.make_async_copy(h_hbm.at[ib.at[0]], rows, sg)

        def scatter_add(ib, rows):
            pltpu.sync_copy(rows, agg_sh.at[ib.at[1]], add=True)

        @pl.loop(0, _RCHUNK)
        def _(i):
            @pl.loop(0, D, step=16)
            def _(j):
                zbuf[i, pl.ds(j, 16)] = jnp.zeros((16,), jnp.float32)

        def zero_rows(c, n):
            pltpu.sync_copy(zbuf.at[pl.ds(0, n)],
                            agg_sh.at[pl.ds(c * _RCHUNK, n)])

        @pl.loop(0, oc_full)
        def _(t):
            zero_rows(t * _NS + sid, _RCHUNK)

        @pl.when(sid < oc_rem)
        def _():
            zero_rows(oc_full * _NS + sid, _RCHUNK)

        @pl.when(sid == _NS - 1)
        def _():
            zero_rows(n_oc, _SINK)

        plsc.subcore_barrier()

        @pl.loop(0, n_pairs)
        def _(t):
            load_idx((2 * t) * NW + wid, ib0)
            gather(ib0, rows0, sg0).start()
            load_idx((2 * t + 1) * NW + wid, ib1)
            gather(ib1, rows1, sg1).start()
            gather(ib0, rows0, sg0).wait()
            scatter_add(ib0, rows0)
            gather(ib1, rows1, sg1).wait()
            scatter_add(ib1, rows1)

        plsc.subcore_barrier()

        def out_rows(c):
            pltpu.sync_copy(agg_sh.at[pl.ds(c * _RCHUNK, _RCHUNK)],
                            out_hbm.at[cid, pl.ds(c * _RCHUNK, _RCHUNK)])

        @pl.loop(0, oc_full)
        def _(t):
            out_rows(t * _NS + sid)

        @pl.when(sid < oc_rem)
        def _():
            out_rows(oc_full * _NS + sid)

    return k(h, eidx)


def _pre0_tc(x, disease_vec, batch_col, W1_0):
    N, DF = x.shape
    Bg, DD = disease_vec.shape
    Dh = W1_0.shape[1]
    G = N // _ROWS

    def body(x_ref, dis_ref, b_ref, W1_ref, o_ref):
        iota = lax.broadcasted_iota(jnp.int32, (_ROWS, Bg), 1).astype(
            jnp.float32)
        oh = (b_ref[...] == iota).astype(jnp.float32)
        du = jnp.dot(dis_ref[...], W1_ref[DF:, :],
                     preferred_element_type=jnp.float32)
        q = jnp.dot(x_ref[...], W1_ref[:DF, :],
                    preferred_element_type=jnp.float32)
        q = q + jnp.dot(oh, du, preferred_element_type=jnp.float32)
        o_ref[...] = q

    return pl.pallas_call(
        body,
        grid=(G,),
        in_specs=[
            pl.BlockSpec((_ROWS, DF), lambda i: (i, 0)),
            pl.BlockSpec((Bg, DD), lambda i: (0, 0)),
            pl.BlockSpec((_ROWS, 1), lambda i: (i, 0)),
            pl.BlockSpec((DF + DD, Dh), lambda i: (0, 0)),
        ],
        out_specs=pl.BlockSpec((_ROWS, Dh), lambda i: (i, 0)),
        out_shape=jax.ShapeDtypeStruct((N, Dh), jnp.float32),
    )(x, disease_vec, batch_col, W1_0)


def _gin0_mlp_tc(q, agg, scale, b1, W2, b2):
    N, Dh = q.shape
    G = N // _ROWS

    def body(q_ref, a0_ref, a1_ref, s_ref, b1_ref, W2_ref, b2_ref, o_ref):
        z = q_ref[...] * s_ref[...] + a0_ref[0] + a1_ref[0]
        z = jnp.maximum(z + b1_ref[...], 0.0)
        z = jnp.dot(z, W2_ref[...], preferred_element_type=jnp.float32)
        o_ref[...] = jnp.maximum(z + b2_ref[...], 0.0)

    return pl.pallas_call(
        body,
        grid=(G,),
        in_specs=[
            pl.BlockSpec((_ROWS, Dh), lambda i: (i, 0)),
            pl.BlockSpec((1, _ROWS, Dh), lambda i: (0, i, 0)),
            pl.BlockSpec((1, _ROWS, Dh), lambda i: (1, i, 0)),
            pl.BlockSpec((1, 1), lambda i: (0, 0)),
            pl.BlockSpec((1, Dh), lambda i: (0, 0)),
            pl.BlockSpec((Dh, Dh), lambda i: (0, 0)),
            pl.BlockSpec((1, Dh), lambda i: (0, 0)),
        ],
        out_specs=pl.BlockSpec((_ROWS, Dh), lambda i: (i, 0)),
        out_shape=jax.ShapeDtypeStruct((N, Dh), jnp.float32),
    )(q, agg, agg, scale, b1, W2, b2)


def _gin_mlp_tc(h, agg, scale, W1, b1, W2, b2):
    N, Din = h.shape
    Dh = W1.shape[1]
    G = N // _ROWS

    def body(h_ref, a0_ref, a1_ref, s_ref, W1_ref, b1_ref, W2_ref, b2_ref,
             o_ref):
        z = h_ref[...] * s_ref[...] + a0_ref[0] + a1_ref[0]
        z = jnp.dot(z, W1_ref[...], preferred_element_type=jnp.float32)
        z = jnp.maximum(z + b1_ref[...], 0.0)
        z = jnp.dot(z, W2_ref[...], preferred_element_type=jnp.float32)
        o_ref[...] = jnp.maximum(z + b2_ref[...], 0.0)

    return pl.pallas_call(
        body,
        grid=(G,),
        in_specs=[
            pl.BlockSpec((_ROWS, Din), lambda i: (i, 0)),
            pl.BlockSpec((1, _ROWS, Din), lambda i: (0, i, 0)),
            pl.BlockSpec((1, _ROWS, Din), lambda i: (1, i, 0)),
            pl.BlockSpec((1, 1), lambda i: (0, 0)),
            pl.BlockSpec((Din, Dh), lambda i: (0, 0)),
            pl.BlockSpec((1, Dh), lambda i: (0, 0)),
            pl.BlockSpec((Dh, Dh), lambda i: (0, 0)),
            pl.BlockSpec((1, Dh), lambda i: (0, 0)),
        ],
        out_specs=pl.BlockSpec((_ROWS, Dh), lambda i: (i, 0)),
        out_shape=jax.ShapeDtypeStruct((N, Dh), jnp.float32),
    )(h, agg, agg, scale, W1, b1, W2, b2)


def _gin_final_tc(h, agg, scale, W1, b1, W2, b2, batch_row, num_graphs,
                  W_mu, b_mu, W_lv, b_lv):
    N, Din = h.shape
    Dh = W1.shape[1]
    LAT = W_mu.shape[1]
    G = batch_row.shape[0]
    Bg = num_graphs

    def body(h_ref, a0_ref, a1_ref, s_ref, W1_ref, b1_ref, W2_ref, b2_ref,
             b_row_ref, Wmu_ref, bmu_ref, Wlv_ref, blv_ref,
             mu_ref, lv_ref, g_acc):
        i = pl.program_id(0)
        z = h_ref[...] * s_ref[...] + a0_ref[0] + a1_ref[0]
        z = jnp.dot(z, W1_ref[...], preferred_element_type=jnp.float32)
        z = jnp.maximum(z + b1_ref[...], 0.0)
        z = jnp.dot(z, W2_ref[...], preferred_element_type=jnp.float32)
        z = jnp.maximum(z + b2_ref[...], 0.0)
        iota = lax.broadcasted_iota(jnp.int32, (Bg, _ROWS), 0).astype(
            jnp.float32)
        ohT = (b_row_ref[0] == iota).astype(jnp.float32)
        g = jnp.dot(ohT, z, preferred_element_type=jnp.float32)

        @pl.when(i == 0)
        def _():
            g_acc[...] = g

        @pl.when(i > 0)
        def _():
            g_acc[...] = g_acc[...] + g

        @pl.when(i == G - 1)
        def _():
            gg = g_acc[...]
            mu_ref[...] = (jnp.dot(gg, Wmu_ref[...],
                                   preferred_element_type=jnp.float32)
                           + bmu_ref[...])
            lv_ref[...] = (jnp.dot(gg, Wlv_ref[...],
                                   preferred_element_type=jnp.float32)
                           + blv_ref[...])

    return pl.pallas_call(
        body,
        grid=(G,),
        in_specs=[
            pl.BlockSpec((_ROWS, Din), lambda i: (i, 0)),
            pl.BlockSpec((1, _ROWS, Din), lambda i: (0, i, 0)),
            pl.BlockSpec((1, _ROWS, Din), lambda i: (1, i, 0)),
            pl.BlockSpec((1, 1), lambda i: (0, 0)),
            pl.BlockSpec((Din, Dh), lambda i: (0, 0)),
            pl.BlockSpec((1, Dh), lambda i: (0, 0)),
            pl.BlockSpec((Dh, Dh), lambda i: (0, 0)),
            pl.BlockSpec((1, Dh), lambda i: (0, 0)),
            pl.BlockSpec((1, 1, _ROWS), lambda i: (i, 0, 0)),
            pl.BlockSpec((Dh, LAT), lambda i: (0, 0)),
            pl.BlockSpec((1, LAT), lambda i: (0, 0)),
            pl.BlockSpec((Dh, LAT), lambda i: (0, 0)),
            pl.BlockSpec((1, LAT), lambda i: (0, 0)),
        ],
        out_specs=[
            pl.BlockSpec((Bg, LAT), lambda i: (0, 0)),
            pl.BlockSpec((Bg, LAT), lambda i: (0, 0)),
        ],
        out_shape=[
            jax.ShapeDtypeStruct((Bg, LAT), jnp.float32),
            jax.ShapeDtypeStruct((Bg, LAT), jnp.float32),
        ],
        scratch_shapes=[pltpu.VMEM((Bg, Dh), jnp.float32)],
    )(h, agg, agg, scale, W1, b1, W2, b2, batch_row,
      W_mu, b_mu, W_lv, b_lv)


def kernel(x, edge_index, batch, disease_vec,
           W1_0, b1_0, W2_0, b2_0, eps_0,
           W1_1, b1_1, W2_1, b2_1, eps_1,
           W1_2, b1_2, W2_2, b2_2, eps_2,
           W_mu, b_mu, W_lv, b_lv):
    N = x.shape[0]
    E = edge_index.shape[1]
    E_pad = _NC * _NS * _CPW * _CH
    assert E_pad >= E
    pad = jnp.stack([jnp.zeros((E_pad - E,), jnp.int32),
                     N + (jnp.arange(E_pad - E, dtype=jnp.int32) % _SINK)])
    eidx = jnp.concatenate([edge_index, pad], axis=1)
    batch_f = batch.astype(jnp.float32)
    batch_col = batch_f.reshape(N, 1)
    G = N // _ROWS
    batch_row = batch_f.reshape(G, 1, _ROWS)

    q = _pre0_tc(x, disease_vec, batch_col, W1_0)
    agg = _segment_sum_sc(q, eidx)
    h = _gin0_mlp_tc(q, agg, (1.0 + eps_0).reshape(1, 1),
                     b1_0.reshape(1, -1), W2_0, b2_0.reshape(1, -1))

    agg = _segment_sum_sc(h, eidx)
    h = _gin_mlp_tc(h, agg, (1.0 + eps_1).reshape(1, 1),
                    W1_1, b1_1.reshape(1, -1), W2_1, b2_1.reshape(1, -1))

    agg = _segment_sum_sc(h, eidx)
    mu, lv = _gin_final_tc(
        h, agg, (1.0 + eps_2).reshape(1, 1), W1_2, b1_2.reshape(1, -1),
        W2_2, b2_2.reshape(1, -1), batch_row, disease_vec.shape[0],
        W_mu, b_mu.reshape(1, -1), W_lv, b_lv.reshape(1, -1))
    return (mu, lv)


# speedup vs baseline: 2.5227x; 2.5227x over previous
import functools

import jax
import jax.numpy as jnp
from jax import lax
from jax.experimental import pallas as pl
from jax.experimental.pallas import tpu as pltpu
from jax.experimental.pallas import tpu_sc as plsc

_NC = 2
_NS = 16
_ROWS = 400


_CH = 128
_CPW = 80
_RCHUNK = 80


_SINK = 80


def _segment_sum_sc(h, eidx):
    N, D = h.shape
    E = eidx.shape[1]
    NW = _NC * _NS
    n_ec = E // _CH
    assert n_ec == NW * _CPW and n_ec * _CH == E
    NP = N + _SINK
    n_oc = N // _RCHUNK
    assert n_oc * _RCHUNK == N and D % 16 == 0
    oc_full, oc_rem = divmod(n_oc, _NS)
    n_pairs = _CPW // 2
    mesh = plsc.VectorSubcoreMesh(core_axis_name="c", subcore_axis_name="s")

    @functools.partial(
        pl.kernel,
        out_type=jax.ShapeDtypeStruct((_NC, N, D), jnp.float32),
        mesh=mesh,
        scratch_types=[
            pltpu.VMEM((2, _CH), jnp.int32),
            pltpu.VMEM((2, _CH), jnp.int32),
            pltpu.VMEM((_CH, D), jnp.float32),
            pltpu.VMEM((_CH, D), jnp.float32),
            pltpu.VMEM((_RCHUNK, D), jnp.float32),
            pltpu.VMEM_SHARED((NP, D), jnp.float32),
            pltpu.SemaphoreType.DMA,
            pltpu.SemaphoreType.DMA,
        ],
    )
    def k(h_hbm, eidx_hbm, out_hbm, ib0, ib1, rows0, rows1, zbuf, agg_sh,
          sg0, sg1):
        cid = lax.axis_index("c")
        sid = lax.axis_index("s")
        wid = cid * _NS + sid

        def load_idx(c, ib):
            pltpu.sync_copy(eidx_hbm.at[:, pl.ds(c * _CH, _CH)], ib)

        def gather(ib, rows, sg):
            return pltpu.make_async_copy(h_hbm.at[ib.at[0]], rows, sg)

        def scatter_add(ib, rows):
            pltpu.sync_copy(rows, agg_sh.at[ib.at[1]], add=True)

        @pl.loop(0, _RCHUNK)
        def _(i):
            @pl.loop(0, D, step=16)
            def _(j):
                zbuf[i, pl.ds(j, 16)] = jnp.zeros((16,), jnp.float32)

        def zero_rows(c, n):
            pltpu.sync_copy(zbuf.at[pl.ds(0, n)],
                            agg_sh.at[pl.ds(c * _RCHUNK, n)])

        @pl.loop(0, oc_full)
        def _(t):
            zero_rows(t * _NS + sid, _RCHUNK)

        @pl.when(sid < oc_rem)
        def _():
            zero_rows(oc_full * _NS + sid, _RCHUNK)

        @pl.when(sid == _NS - 1)
        def _():
            zero_rows(n_oc, _SINK)

        plsc.subcore_barrier()

        @pl.loop(0, n_pairs)
        def _(t):
            load_idx((2 * t) * NW + wid, ib0)
            gather(ib0, rows0, sg0).start()
            load_idx((2 * t + 1) * NW + wid, ib1)
            gather(ib1, rows1, sg1).start()
            gather(ib0, rows0, sg0).wait()
            scatter_add(ib0, rows0)
            gather(ib1, rows1, sg1).wait()
            scatter_add(ib1, rows1)

        plsc.subcore_barrier()

        def out_rows(c):
            pltpu.sync_copy(agg_sh.at[pl.ds(c * _RCHUNK, _RCHUNK)],
                            out_hbm.at[cid, pl.ds(c * _RCHUNK, _RCHUNK)])

        @pl.loop(0, oc_full)
        def _(t):
            out_rows(t * _NS + sid)

        @pl.when(sid < oc_rem)
        def _():
            out_rows(oc_full * _NS + sid)

    return k(h, eidx)


def _pre0_tc(x, disease_vec, batch_col, W1_0):
    N, DF = x.shape
    Bg, DD = disease_vec.shape
    Dh = W1_0.shape[1]
    G = N // _ROWS

    def body(x_ref, dis_ref, b_ref, W1_ref, o_ref):
        iota = lax.broadcasted_iota(jnp.int32, (_ROWS, Bg), 1).astype(
            jnp.float32)
        oh = (b_ref[...] == iota).astype(jnp.float32)
        du = jnp.dot(dis_ref[...], W1_ref[DF:, :],
                     preferred_element_type=jnp.float32)
        q = jnp.dot(x_ref[...], W1_ref[:DF, :],
                    preferred_element_type=jnp.float32)
        q = q + jnp.dot(oh, du, preferred_element_type=jnp.float32)
        o_ref[...] = q

    return pl.pallas_call(
        body,
        grid=(G,),
        in_specs=[
            pl.BlockSpec((_ROWS, DF), lambda i: (i, 0)),
            pl.BlockSpec((Bg, DD), lambda i: (0, 0)),
            pl.BlockSpec((_ROWS, 1), lambda i: (i, 0)),
            pl.BlockSpec((DF + DD, Dh), lambda i: (0, 0)),
        ],
        out_specs=pl.BlockSpec((_ROWS, Dh), lambda i: (i, 0)),
        out_shape=jax.ShapeDtypeStruct((N, Dh), jnp.float32),
    )(x, disease_vec, batch_col, W1_0)


def _gin0_mlp_tc(q, agg, scale, b1, W2, b2):
    N, Dh = q.shape
    G = N // _ROWS

    def body(q_ref, a0_ref, a1_ref, s_ref, b1_ref, W2_ref, b2_ref, o_ref):
        z = q_ref[...] * s_ref[...] + a0_ref[0] + a1_ref[0]
        z = jnp.maximum(z + b1_ref[...], 0.0)
        z = jnp.dot(z, W2_ref[...], preferred_element_type=jnp.float32)
        o_ref[...] = jnp.maximum(z + b2_ref[...], 0.0)

    return pl.pallas_call(
        body,
        grid=(G,),
        in_specs=[
            pl.BlockSpec((_ROWS, Dh), lambda i: (i, 0)),
            pl.BlockSpec((1, _ROWS, Dh), lambda i: (0, i, 0)),
            pl.BlockSpec((1, _ROWS, Dh), lambda i: (1, i, 0)),
            pl.BlockSpec((1, 1), lambda i: (0, 0)),
            pl.BlockSpec((1, Dh), lambda i: (0, 0)),
            pl.BlockSpec((Dh, Dh), lambda i: (0, 0)),
            pl.BlockSpec((1, Dh), lambda i: (0, 0)),
        ],
        out_specs=pl.BlockSpec((_ROWS, Dh), lambda i: (i, 0)),
        out_shape=jax.ShapeDtypeStruct((N, Dh), jnp.float32),
    )(q, agg, agg, scale, b1, W2, b2)


def _gin_mlp_tc(h, agg, scale, W1, b1, W2, b2):
    N, Din = h.shape
    Dh = W1.shape[1]
    G = N // _ROWS

    def body(h_ref, a0_ref, a1_ref, s_ref, W1_ref, b1_ref, W2_ref, b2_ref,
             o_ref):
        z = h_ref[...] * s_ref[...] + a0_ref[0] + a1_ref[0]
        z = jnp.dot(z, W1_ref[...], preferred_element_type=jnp.float32)
        z = jnp.maximum(z + b1_ref[...], 0.0)
        z = jnp.dot(z, W2_ref[...], preferred_element_type=jnp.float32)
        o_ref[...] = jnp.maximum(z + b2_ref[...], 0.0)

    return pl.pallas_call(
        body,
        grid=(G,),
        in_specs=[
            pl.BlockSpec((_ROWS, Din), lambda i: (i, 0)),
            pl.BlockSpec((1, _ROWS, Din), lambda i: (0, i, 0)),
            pl.BlockSpec((1, _ROWS, Din), lambda i: (1, i, 0)),
            pl.BlockSpec((1, 1), lambda i: (0, 0)),
            pl.BlockSpec((Din, Dh), lambda i: (0, 0)),
            pl.BlockSpec((1, Dh), lambda i: (0, 0)),
            pl.BlockSpec((Dh, Dh), lambda i: (0, 0)),
            pl.BlockSpec((1, Dh), lambda i: (0, 0)),
        ],
        out_specs=pl.BlockSpec((_ROWS, Dh), lambda i: (i, 0)),
        out_shape=jax.ShapeDtypeStruct((N, Dh), jnp.float32),
    )(h, agg, agg, scale, W1, b1, W2, b2)


def _gin_final_tc(h, agg, scale, W1, b1, W2, b2, batch_row, num_graphs,
                  W_mu, b_mu, W_lv, b_lv):
    N, Din = h.shape
    Dh = W1.shape[1]
    LAT = W_mu.shape[1]
    G = batch_row.shape[0]
    Bg = num_graphs

    def body(h_ref, a0_ref, a1_ref, s_ref, W1_ref, b1_ref, W2_ref, b2_ref,
             b_row_ref, Wmu_ref, bmu_ref, Wlv_ref, blv_ref,
             mu_ref, lv_ref, g_acc):
        i = pl.program_id(0)
        z = h_ref[...] * s_ref[...] + a0_ref[0] + a1_ref[0]
        z = jnp.dot(z, W1_ref[...], preferred_element_type=jnp.float32)
        z = jnp.maximum(z + b1_ref[...], 0.0)
        z = jnp.dot(z, W2_ref[...], preferred_element_type=jnp.float32)
        z = jnp.maximum(z + b2_ref[...], 0.0)
        iota = lax.broadcasted_iota(jnp.int32, (Bg, _ROWS), 0).astype(
            jnp.float32)
        ohT = (b_row_ref[0] == iota).astype(jnp.float32)
        g = jnp.dot(ohT, z, preferred_element_type=jnp.float32)

        @pl.when(i == 0)
        def _():
            g_acc[...] = g

        @pl.when(i > 0)
        def _():
            g_acc[...] = g_acc[...] + g

        @pl.when(i == G - 1)
        def _():
            gg = g_acc[...]
            mu_ref[...] = (jnp.dot(gg, Wmu_ref[...],
                                   preferred_element_type=jnp.float32)
                           + bmu_ref[...])
            lv_ref[...] = (jnp.dot(gg, Wlv_ref[...],
                                   preferred_element_type=jnp.float32)
                           + blv_ref[...])

    return pl.pallas_call(
        body,
        grid=(G,),
        in_specs=[
            pl.BlockSpec((_ROWS, Din), lambda i: (i, 0)),
            pl.BlockSpec((1, _ROWS, Din), lambda i: (0, i, 0)),
            pl.BlockSpec((1, _ROWS, Din), lambda i: (1, i, 0)),
            pl.BlockSpec((1, 1), lambda i: (0, 0)),
            pl.BlockSpec((Din, Dh), lambda i: (0, 0)),
            pl.BlockSpec((1, Dh), lambda i: (0, 0)),
            pl.BlockSpec((Dh, Dh), lambda i: (0, 0)),
            pl.BlockSpec((1, Dh), lambda i: (0, 0)),
            pl.BlockSpec((1, 1, _ROWS), lambda i: (i, 0, 0)),
            pl.BlockSpec((Dh, LAT), lambda i: (0, 0)),
            pl.BlockSpec((1, LAT), lambda i: (0, 0)),
            pl.BlockSpec((Dh, LAT), lambda i: (0, 0)),
            pl.BlockSpec((1, LAT), lambda i: (0, 0)),
        ],
        out_specs=[
            pl.BlockSpec((Bg, LAT), lambda i: (0, 0)),
            pl.BlockSpec((Bg, LAT), lambda i: (0, 0)),
        ],
        out_shape=[
            jax.ShapeDtypeStruct((Bg, LAT), jnp.float32),
            jax.ShapeDtypeStruct((Bg, LAT), jnp.float32),
        ],
        scratch_shapes=[pltpu.VMEM((Bg, Dh), jnp.float32)],
    )(h, agg, agg, scale, W1, b1, W2, b2, batch_row,
      W_mu, b_mu, W_lv, b_lv)


def kernel(x, edge_index, batch, disease_vec,
           W1_0, b1_0, W2_0, b2_0, eps_0,
           W1_1, b1_1, W2_1, b2_1, eps_1,
           W1_2, b1_2, W2_2, b2_2, eps_2,
           W_mu, b_mu, W_lv, b_lv):
    N = x.shape[0]
    E = edge_index.shape[1]
    E_pad = _NC * _NS * _CPW * _CH
    assert E_pad >= E
    ar = jnp.arange(E_pad - E, dtype=jnp.int32)
    pad = jnp.stack([(ar * 37) % N, N + (ar % _SINK)])
    eidx = jnp.concatenate([edge_index, pad], axis=1)
    batch_f = batch.astype(jnp.float32)
    batch_col = batch_f.reshape(N, 1)
    G = N // _ROWS
    batch_row = batch_f.reshape(G, 1, _ROWS)

    q = _pre0_tc(x, disease_vec, batch_col, W1_0)
    agg = _segment_sum_sc(q, eidx)
    h = _gin0_mlp_tc(q, agg, (1.0 + eps_0).reshape(1, 1),
                     b1_0.reshape(1, -1), W2_0, b2_0.reshape(1, -1))

    agg = _segment_sum_sc(h, eidx)
    h = _gin_mlp_tc(h, agg, (1.0 + eps_1).reshape(1, 1),
                    W1_1, b1_1.reshape(1, -1), W2_1, b2_1.reshape(1, -1))

    agg = _segment_sum_sc(h, eidx)
    mu, lv = _gin_final_tc(
        h, agg, (1.0 + eps_2).reshape(1, 1), W1_2, b1_2.reshape(1, -1),
        W2_2, b2_2.reshape(1, -1), batch_row, disease_vec.shape[0],
        W_mu, b_mu.reshape(1, -1), W_lv, b_lv.reshape(1, -1))
    return (mu, lv)


# speedup vs baseline: 3.0293x; 1.2008x over previous
import functools

import jax
import jax.numpy as jnp
from jax import lax
from jax.experimental import pallas as pl
from jax.experimental.pallas import tpu as pltpu
from jax.experimental.pallas import tpu_sc as plsc

_NC = 2
_NS = 16
_ROWS = 400


_CH = 128
_CPW = 80
_RCHUNK = 80


_SINK = 80


def _segment_sum_sc(h, eidx):
    N, D = h.shape
    E = eidx.shape[1]
    NW = _NC * _NS
    n_ec = E // _CH
    assert n_ec == NW * _CPW and n_ec * _CH == E
    NP = N + _SINK
    n_oc = N // _RCHUNK
    assert n_oc * _RCHUNK == N and D % 16 == 0
    oc_full, oc_rem = divmod(n_oc, _NS)
    n_pairs = _CPW // 2
    mesh = plsc.VectorSubcoreMesh(core_axis_name="c", subcore_axis_name="s")

    @functools.partial(
        pl.kernel,
        out_type=jax.ShapeDtypeStruct((_NC, N, D), jnp.float32),
        mesh=mesh,
        scratch_types=[
            pltpu.VMEM((2, _CH), jnp.int32),
            pltpu.VMEM((2, _CH), jnp.int32),
            pltpu.VMEM((_CH, D), jnp.float32),
            pltpu.VMEM((_CH, D), jnp.float32),
            pltpu.VMEM((_RCHUNK, D), jnp.float32),
            pltpu.VMEM_SHARED((NP, D), jnp.float32),
            pltpu.SemaphoreType.DMA,
            pltpu.SemaphoreType.DMA,
        ],
    )
    def k(h_hbm, eidx_hbm, out_hbm, ib0, ib1, rows0, rows1, zbuf, agg_sh,
          sg0, sg1):
        cid = lax.axis_index("c")
        sid = lax.axis_index("s")
        wid = cid * _NS + sid

        def load_idx(c, ib):
            pltpu.sync_copy(eidx_hbm.at[:, pl.ds(c * _CH, _CH)], ib)

        def gather(ib, rows, sg):
            return pltpu.make_async_copy(h_hbm.at[ib.at[0]], rows, sg)

        def scatter_add(ib, rows):
            pltpu.sync_copy(rows, agg_sh.at[ib.at[1]], add=True)

        load_idx(wid, ib0)
        gather(ib0, rows0, sg0).start()
        load_idx(NW + wid, ib1)
        gather(ib1, rows1, sg1).start()

        @pl.loop(0, _RCHUNK)
        def _(i):
            @pl.loop(0, D, step=16)
            def _(j):
                zbuf[i, pl.ds(j, 16)] = jnp.zeros((16,), jnp.float32)

        def zero_rows(c, n):
            pltpu.sync_copy(zbuf.at[pl.ds(0, n)],
                            agg_sh.at[pl.ds(c * _RCHUNK, n)])

        @pl.loop(0, oc_full)
        def _(t):
            zero_rows(t * _NS + sid, _RCHUNK)

        @pl.when(sid < oc_rem)
        def _():
            zero_rows(oc_full * _NS + sid, _RCHUNK)

        @pl.when(sid == _NS - 1)
        def _():
            zero_rows(n_oc, _SINK)

        plsc.subcore_barrier()

        @pl.loop(0, n_pairs)
        def _(t):
            gather(ib0, rows0, sg0).wait()
            scatter_add(ib0, rows0)

            @pl.when(t < n_pairs - 1)
            def _():
                load_idx((2 * t + 2) * NW + wid, ib0)
                gather(ib0, rows0, sg0).start()

            gather(ib1, rows1, sg1).wait()
            scatter_add(ib1, rows1)

            @pl.when(t < n_pairs - 1)
            def _():
                load_idx((2 * t + 3) * NW + wid, ib1)
                gather(ib1, rows1, sg1).start()

        plsc.subcore_barrier()

        def out_rows(c):
            pltpu.sync_copy(agg_sh.at[pl.ds(c * _RCHUNK, _RCHUNK)],
                            out_hbm.at[cid, pl.ds(c * _RCHUNK, _RCHUNK)])

        @pl.loop(0, oc_full)
        def _(t):
            out_rows(t * _NS + sid)

        @pl.when(sid < oc_rem)
        def _():
            out_rows(oc_full * _NS + sid)

    return k(h, eidx)


def _pre0_tc(x, disease_vec, batch_col, W1_0):
    N, DF = x.shape
    Bg, DD = disease_vec.shape
    Dh = W1_0.shape[1]
    G = N // _ROWS

    def body(x_ref, dis_ref, b_ref, W1_ref, o_ref):
        iota = lax.broadcasted_iota(jnp.int32, (_ROWS, Bg), 1).astype(
            jnp.float32)
        oh = (b_ref[...] == iota).astype(jnp.float32)
        du = jnp.dot(dis_ref[...], W1_ref[DF:, :],
                     preferred_element_type=jnp.float32)
        q = jnp.dot(x_ref[...], W1_ref[:DF, :],
                    preferred_element_type=jnp.float32)
        q = q + jnp.dot(oh, du, preferred_element_type=jnp.float32)
        o_ref[...] = q

    return pl.pallas_call(
        body,
        grid=(G,),
        in_specs=[
            pl.BlockSpec((_ROWS, DF), lambda i: (i, 0)),
            pl.BlockSpec((Bg, DD), lambda i: (0, 0)),
            pl.BlockSpec((_ROWS, 1), lambda i: (i, 0)),
            pl.BlockSpec((DF + DD, Dh), lambda i: (0, 0)),
        ],
        out_specs=pl.BlockSpec((_ROWS, Dh), lambda i: (i, 0)),
        out_shape=jax.ShapeDtypeStruct((N, Dh), jnp.float32),
    )(x, disease_vec, batch_col, W1_0)


def _gin0_mlp_tc(q, agg, scale, b1, W2, b2):
    N, Dh = q.shape
    G = N // _ROWS

    def body(q_ref, a0_ref, a1_ref, s_ref, b1_ref, W2_ref, b2_ref, o_ref):
        z = q_ref[...] * s_ref[...] + a0_ref[0] + a1_ref[0]
        z = jnp.maximum(z + b1_ref[...], 0.0)
        z = jnp.dot(z, W2_ref[...], preferred_element_type=jnp.float32)
        o_ref[...] = jnp.maximum(z + b2_ref[...], 0.0)

    return pl.pallas_call(
        body,
        grid=(G,),
        in_specs=[
            pl.BlockSpec((_ROWS, Dh), lambda i: (i, 0)),
            pl.BlockSpec((1, _ROWS, Dh), lambda i: (0, i, 0)),
            pl.BlockSpec((1, _ROWS, Dh), lambda i: (1, i, 0)),
            pl.BlockSpec((1, 1), lambda i: (0, 0)),
            pl.BlockSpec((1, Dh), lambda i: (0, 0)),
            pl.BlockSpec((Dh, Dh), lambda i: (0, 0)),
            pl.BlockSpec((1, Dh), lambda i: (0, 0)),
        ],
        out_specs=pl.BlockSpec((_ROWS, Dh), lambda i: (i, 0)),
        out_shape=jax.ShapeDtypeStruct((N, Dh), jnp.float32),
    )(q, agg, agg, scale, b1, W2, b2)


def _gin_mlp_tc(h, agg, scale, W1, b1, W2, b2):
    N, Din = h.shape
    Dh = W1.shape[1]
    G = N // _ROWS

    def body(h_ref, a0_ref, a1_ref, s_ref, W1_ref, b1_ref, W2_ref, b2_ref,
             o_ref):
        z = h_ref[...] * s_ref[...] + a0_ref[0] + a1_ref[0]
        z = jnp.dot(z, W1_ref[...], preferred_element_type=jnp.float32)
        z = jnp.maximum(z + b1_ref[...], 0.0)
        z = jnp.dot(z, W2_ref[...], preferred_element_type=jnp.float32)
        o_ref[...] = jnp.maximum(z + b2_ref[...], 0.0)

    return pl.pallas_call(
        body,
        grid=(G,),
        in_specs=[
            pl.BlockSpec((_ROWS, Din), lambda i: (i, 0)),
            pl.BlockSpec((1, _ROWS, Din), lambda i: (0, i, 0)),
            pl.BlockSpec((1, _ROWS, Din), lambda i: (1, i, 0)),
            pl.BlockSpec((1, 1), lambda i: (0, 0)),
            pl.BlockSpec((Din, Dh), lambda i: (0, 0)),
            pl.BlockSpec((1, Dh), lambda i: (0, 0)),
            pl.BlockSpec((Dh, Dh), lambda i: (0, 0)),
            pl.BlockSpec((1, Dh), lambda i: (0, 0)),
        ],
        out_specs=pl.BlockSpec((_ROWS, Dh), lambda i: (i, 0)),
        out_shape=jax.ShapeDtypeStruct((N, Dh), jnp.float32),
    )(h, agg, agg, scale, W1, b1, W2, b2)


def _gin_final_tc(h, agg, scale, W1, b1, W2, b2, batch_row, num_graphs,
                  W_mu, b_mu, W_lv, b_lv):
    N, Din = h.shape
    Dh = W1.shape[1]
    LAT = W_mu.shape[1]
    G = batch_row.shape[0]
    Bg = num_graphs

    def body(h_ref, a0_ref, a1_ref, s_ref, W1_ref, b1_ref, W2_ref, b2_ref,
             b_row_ref, Wmu_ref, bmu_ref, Wlv_ref, blv_ref,
             mu_ref, lv_ref, g_acc):
        i = pl.program_id(0)
        z = h_ref[...] * s_ref[...] + a0_ref[0] + a1_ref[0]
        z = jnp.dot(z, W1_ref[...], preferred_element_type=jnp.float32)
        z = jnp.maximum(z + b1_ref[...], 0.0)
        z = jnp.dot(z, W2_ref[...], preferred_element_type=jnp.float32)
        z = jnp.maximum(z + b2_ref[...], 0.0)
        iota = lax.broadcasted_iota(jnp.int32, (Bg, _ROWS), 0).astype(
            jnp.float32)
        ohT = (b_row_ref[0] == iota).astype(jnp.float32)
        g = jnp.dot(ohT, z, preferred_element_type=jnp.float32)

        @pl.when(i == 0)
        def _():
            g_acc[...] = g

        @pl.when(i > 0)
        def _():
            g_acc[...] = g_acc[...] + g

        @pl.when(i == G - 1)
        def _():
            gg = g_acc[...]
            mu_ref[...] = (jnp.dot(gg, Wmu_ref[...],
                                   preferred_element_type=jnp.float32)
                           + bmu_ref[...])
            lv_ref[...] = (jnp.dot(gg, Wlv_ref[...],
                                   preferred_element_type=jnp.float32)
                           + blv_ref[...])

    return pl.pallas_call(
        body,
        grid=(G,),
        in_specs=[
            pl.BlockSpec((_ROWS, Din), lambda i: (i, 0)),
            pl.BlockSpec((1, _ROWS, Din), lambda i: (0, i, 0)),
            pl.BlockSpec((1, _ROWS, Din), lambda i: (1, i, 0)),
            pl.BlockSpec((1, 1), lambda i: (0, 0)),
            pl.BlockSpec((Din, Dh), lambda i: (0, 0)),
            pl.BlockSpec((1, Dh), lambda i: (0, 0)),
            pl.BlockSpec((Dh, Dh), lambda i: (0, 0)),
            pl.BlockSpec((1, Dh), lambda i: (0, 0)),
            pl.BlockSpec((1, 1, _ROWS), lambda i: (i, 0, 0)),
            pl.BlockSpec((Dh, LAT), lambda i: (0, 0)),
            pl.BlockSpec((1, LAT), lambda i: (0, 0)),
            pl.BlockSpec((Dh, LAT), lambda i: (0, 0)),
            pl.BlockSpec((1, LAT), lambda i: (0, 0)),
        ],
        out_specs=[
            pl.BlockSpec((Bg, LAT), lambda i: (0, 0)),
            pl.BlockSpec((Bg, LAT), lambda i: (0, 0)),
        ],
        out_shape=[
            jax.ShapeDtypeStruct((Bg, LAT), jnp.float32),
            jax.ShapeDtypeStruct((Bg, LAT), jnp.float32),
        ],
        scratch_shapes=[pltpu.VMEM((Bg, Dh), jnp.float32)],
    )(h, agg, agg, scale, W1, b1, W2, b2, batch_row,
      W_mu, b_mu, W_lv, b_lv)


def kernel(x, edge_index, batch, disease_vec,
           W1_0, b1_0, W2_0, b2_0, eps_0,
           W1_1, b1_1, W2_1, b2_1, eps_1,
           W1_2, b1_2, W2_2, b2_2, eps_2,
           W_mu, b_mu, W_lv, b_lv):
    N = x.shape[0]
    E = edge_index.shape[1]
    E_pad = _NC * _NS * _CPW * _CH
    assert E_pad >= E
    ar = jnp.arange(E_pad - E, dtype=jnp.int32)
    pad = jnp.stack([(ar * 37) % N, N + (ar % _SINK)])
    eidx = jnp.concatenate([edge_index, pad], axis=1)
    batch_f = batch.astype(jnp.float32)
    batch_col = batch_f.reshape(N, 1)
    G = N // _ROWS
    batch_row = batch_f.reshape(G, 1, _ROWS)

    q = _pre0_tc(x, disease_vec, batch_col, W1_0)
    agg = _segment_sum_sc(q, eidx)
    h = _gin0_mlp_tc(q, agg, (1.0 + eps_0).reshape(1, 1),
                     b1_0.reshape(1, -1), W2_0, b2_0.reshape(1, -1))

    agg = _segment_sum_sc(h, eidx)
    h = _gin_mlp_tc(h, agg, (1.0 + eps_1).reshape(1, 1),
                    W1_1, b1_1.reshape(1, -1), W2_1, b2_1.reshape(1, -1))

    agg = _segment_sum_sc(h, eidx)
    mu, lv = _gin_final_tc(
        h, agg, (1.0 + eps_2).reshape(1, 1), W1_2, b1_2.reshape(1, -1),
        W2_2, b2_2.reshape(1, -1), batch_row, disease_vec.shape[0],
        W_mu, b_mu.reshape(1, -1), W_lv, b_lv.reshape(1, -1))
    return (mu, lv)
